# Initial kernel scaffold; baseline (speedup 1.0000x reference)
import jax
import jax.numpy as jnp
from jax.experimental import pallas as pl


def kernel(h, pos_info_0, pos_info_1, pos_info_2, W, b):
    raise NotImplementedError("write your pallas kernel here")


# speedup vs baseline: 1.9817x; 1.9817x over previous
import functools

import jax
import jax.numpy as jnp
from jax import lax
from jax.experimental import pallas as pl
from jax.experimental.pallas import tpu as pltpu
from jax.experimental.pallas import tpu_sc as plsc

N = 50000
D = 512
B = 16384
K1 = 4
K2 = 16
K = 1 + K1 + K2
NC = 2
NS = 16
NW = NC * NS
BPW = B // NW
CB = 8
NCHUNK = BPW // CB
LANES = 16


def _sc_pool(h, idx_flat):
    mesh = plsc.VectorSubcoreMesh(core_axis_name="c", subcore_axis_name="s")

    @functools.partial(
        pl.kernel,
        mesh=mesh,
        out_type=jax.ShapeDtypeStruct((B, 3 * D), jnp.float32),
        scratch_types=[
            pltpu.VMEM((BPW * K,), jnp.int32),
            pltpu.VMEM((CB * K, D), jnp.float32),
            pltpu.VMEM((CB, 3 * D), jnp.float32),
            pltpu.SemaphoreType.DMA,
        ],
    )
    def kern(h_hbm, idx_hbm, out_hbm, idx_v, rows_v, acc_v, sem):
        wid = lax.axis_index("s") * NC + lax.axis_index("c")
        base_b = wid * BPW
        pltpu.sync_copy(idx_hbm.at[pl.ds(base_b * K, BPW * K)], idx_v)

        @pl.loop(0, NCHUNK)
        def _chunk(ch):
            pltpu.async_copy(
                h_hbm.at[idx_v.at[pl.ds(ch * (CB * K), CB * K)]], rows_v, sem
            ).wait()

            @pl.loop(0, D // LANES)
            def _col(j):
                col = j * LANES
                for c in range(CB):
                    r = c * K

                    def ld(i):
                        return rows_v[r + i, pl.ds(col, LANES)]

                    acc_v[c, pl.ds(col, LANES)] = ld(0)
                    s1 = (ld(1) + ld(2)) + (ld(3) + ld(4))
                    acc_v[c, pl.ds(col + D, LANES)] = s1 * 0.25
                    s2a = ((ld(5) + ld(6)) + (ld(7) + ld(8)))
                    s2b = ((ld(9) + ld(10)) + (ld(11) + ld(12)))
                    s2c = ((ld(13) + ld(14)) + (ld(15) + ld(16)))
                    s2d = ((ld(17) + ld(18)) + (ld(19) + ld(20)))
                    s2 = (s2a + s2b) + (s2c + s2d)
                    acc_v[c, pl.ds(col + 2 * D, LANES)] = s2 * (1.0 / 16.0)

            pltpu.sync_copy(acc_v, out_hbm.at[pl.ds(base_b + ch * CB, CB)])

    return kern(h, idx_flat)


def _tc_matmul(cat, wt, b2):
    BT = 1024

    def kern(cat_ref, wt_ref, b_ref, o_ref):
        o_ref[...] = (
            jnp.dot(cat_ref[...], wt_ref[...], preferred_element_type=jnp.float32)
            + b_ref[...]
        )

    return pl.pallas_call(
        kern,
        grid=(B // BT,),
        in_specs=[
            pl.BlockSpec((BT, 3 * D), lambda i: (i, 0)),
            pl.BlockSpec((3 * D, D), lambda i: (0, 0)),
            pl.BlockSpec((1, D), lambda i: (0, 0)),
        ],
        out_specs=pl.BlockSpec((BT, D), lambda i: (i, 0)),
        out_shape=jax.ShapeDtypeStruct((B, D), jnp.float32),
        compiler_params=pltpu.CompilerParams(
            dimension_semantics=("arbitrary",),
        ),
    )(cat, wt, b2)


@jax.jit
def kernel(h, pos_info_0, pos_info_1, pos_info_2, W, b):
    idx_flat = jnp.concatenate(
        [pos_info_0[:, None], pos_info_1, pos_info_2], axis=1
    ).astype(jnp.int32).reshape(-1)
    cat = _sc_pool(h, idx_flat)
    return _tc_matmul(cat, W.T, b[None, :])
